# baseline (speedup 1.0000x reference)
import functools

import jax
import jax.numpy as jnp
from jax import lax
from jax.experimental import pallas as pl
from jax.experimental.pallas import tpu as pltpu
from jax.experimental.pallas import tpu_sc as plsc

VOCAB = 100000
EMBED = 128
HIDDEN = 512
OUT = 256
BATCH = 4096
HIST = 50

NC = 2
NS = 16
NW = NC * NS
ROWS_PER_W = BATCH // NW
CB = 2
CHUNK_IDX = CB * HIST
NCHUNK = ROWS_PER_W // CB
LANES = 16
EV = EMBED // LANES

_sc_mesh = plsc.VectorSubcoreMesh(
    core_axis_name="c", subcore_axis_name="s", num_cores=NC, num_subcores=NS
)


@functools.partial(
    pl.kernel,
    out_type=jax.ShapeDtypeStruct((BATCH, EMBED), jnp.float32),
    mesh=_sc_mesh,
    scratch_types=[
        pltpu.VMEM((NCHUNK, CHUNK_IDX), jnp.int32),
        pltpu.VMEM((NCHUNK, CHUNK_IDX), jnp.int32),
        pltpu.VMEM((CHUNK_IDX, EMBED), jnp.float32),
        pltpu.VMEM((CHUNK_IDX, EMBED), jnp.float32),
        pltpu.VMEM((CHUNK_IDX, EMBED), jnp.float32),
        pltpu.VMEM((CHUNK_IDX, EMBED), jnp.float32),
        pltpu.VMEM((ROWS_PER_W, EMBED), jnp.float32),
        pltpu.VMEM_SHARED((NS * ROWS_PER_W, EMBED), jnp.float32),
        pltpu.SemaphoreType.DMA,
        pltpu.SemaphoreType.DMA,
        pltpu.SemaphoreType.DMA,
        pltpu.SemaphoreType.DMA,
        pltpu.SemaphoreType.DMA,
        pltpu.SemaphoreType.DMA,
        pltpu.SemaphoreType.DMA,
        pltpu.SemaphoreType.DMA,
    ],
)
def _embbag_sum(idx_hbm, bag_hbm, table_hbm, out_hbm, idx_v, bag_v,
                rows0_v, rows1_v, rows2_v, rows3_v, pool_v, spool,
                gsem0, gsem1, gsem2, gsem3, asem0, asem1, asem2, asem3):
    sid = lax.axis_index("s")
    wid = sid * NC + lax.axis_index("c")

    pltpu.sync_copy(idx_hbm.at[pl.ds(wid * NCHUNK, NCHUNK)], idx_v)
    pltpu.sync_copy(bag_hbm.at[sid], bag_v)

    bufs = ((rows0_v, gsem0, asem0), (rows1_v, gsem1, asem1),
            (rows2_v, gsem2, asem2), (rows3_v, gsem3, asem3))
    NBUF = len(bufs)

    def gather(c, buf, gsem):
        pltpu.async_copy(table_hbm.at[idx_v.at[jnp.minimum(c, NCHUNK - 1)]],
                         buf, gsem)

    def drain_gather(buf, gsem):
        pltpu.make_async_copy(table_hbm.at[idx_v.at[0]], buf, gsem).wait()

    gather(0, bufs[0][0], bufs[0][1])
    gather(1, bufs[1][0], bufs[1][1])

    zeros = jnp.zeros((LANES,), jnp.float32)

    def zero_body(r, carry):
        for j in range(EV):
            pool_v[r, pl.ds(j * LANES, LANES)] = zeros
        return carry

    lax.fori_loop(0, ROWS_PER_W, zero_body, 0, unroll=4)
    pltpu.sync_copy(pool_v, spool.at[pl.ds(sid * ROWS_PER_W, ROWS_PER_W)])

    def ring_body(p, carry):
        a = p * NBUF
        for q in range(NBUF):
            buf, gsem, asem = bufs[q]
            nbuf, ngsem, _ = bufs[(q + 2) % NBUF]
            drain_gather(buf, gsem)
            add = pltpu.async_copy(buf, spool.at[bag_v.at[a + q]], asem,
                                   add=True)
            gather(a + q + 2, nbuf, ngsem)
            add.wait()
        return carry

    lax.fori_loop(0, NCHUNK // NBUF, ring_body, 0)
    drain_gather(bufs[0][0], bufs[0][1])
    drain_gather(bufs[1][0], bufs[1][1])

    pltpu.sync_copy(spool.at[pl.ds(sid * ROWS_PER_W, ROWS_PER_W)],
                    out_hbm.at[pl.ds(wid * ROWS_PER_W, ROWS_PER_W)])


def _mlp_body(x_ref, w1_ref, b1_ref, w2_ref, b2_ref, o_ref):
    x = x_ref[...] * (1.0 / HIST)
    h = lax.dot_general(
        x, w1_ref[...], (((1,), (1,)), ((), ())),
        preferred_element_type=jnp.float32,
    ) + b1_ref[...]
    o_ref[...] = lax.dot_general(
        h, w2_ref[...], (((1,), (1,)), ((), ())),
        preferred_element_type=jnp.float32,
    ) + b2_ref[...]


_mlp = pl.pallas_call(
    _mlp_body,
    out_shape=jax.ShapeDtypeStruct((BATCH, OUT), jnp.float32),
)


@jax.jit
def kernel(input_batch, emb_table, W1, b1, W2, b2):
    idx2d = input_batch.astype(jnp.int32).reshape(BATCH // CB, CHUNK_IDX)
    bag3d = (
        jnp.arange(NS, dtype=jnp.int32)[:, None, None] * ROWS_PER_W
        + jnp.arange(NCHUNK, dtype=jnp.int32)[None, :, None] * CB
        + (jnp.arange(CHUNK_IDX, dtype=jnp.int32) // HIST)[None, None, :]
    )
    pooled_sum = _embbag_sum(idx2d, bag3d, emb_table)
    return _mlp(pooled_sum, W1, b1.reshape(1, HIDDEN), W2, b2.reshape(1, OUT))


# speedup vs baseline: 1.2827x; 1.2827x over previous
import functools

import jax
import jax.numpy as jnp
from jax import lax
from jax.experimental import pallas as pl
from jax.experimental.pallas import tpu as pltpu
from jax.experimental.pallas import tpu_sc as plsc

VOCAB = 100000
EMBED = 128
HIDDEN = 512
OUT = 256
BATCH = 4096
HIST = 50

NC = 2
NS = 16
NW = NC * NS
ROWS_PER_W = BATCH // NW
CB = 2
CHUNK_IDX = CB * HIST
NCHUNK = ROWS_PER_W // CB
LANES = 16
EV = EMBED // LANES

_sc_mesh = plsc.VectorSubcoreMesh(
    core_axis_name="c", subcore_axis_name="s", num_cores=NC, num_subcores=NS
)


@functools.partial(
    pl.kernel,
    out_type=jax.ShapeDtypeStruct((BATCH, EMBED), jnp.float32),
    mesh=_sc_mesh,
    scratch_types=[
        pltpu.VMEM((NCHUNK, CHUNK_IDX), jnp.int32),
        pltpu.VMEM((CHUNK_IDX, EMBED), jnp.float32),
        pltpu.VMEM((CHUNK_IDX, EMBED), jnp.float32),
        pltpu.VMEM((CHUNK_IDX, EMBED), jnp.float32),
        pltpu.VMEM((CHUNK_IDX, EMBED), jnp.float32),
        pltpu.VMEM((ROWS_PER_W, EMBED), jnp.float32),
        pltpu.SemaphoreType.DMA,
        pltpu.SemaphoreType.DMA,
        pltpu.SemaphoreType.DMA,
        pltpu.SemaphoreType.DMA,
    ],
)
def _embbag_sum(idx_hbm, table_hbm, out_hbm, idx_v, rows0_v, rows1_v,
                rows2_v, rows3_v, pool_v, sem0, sem1, sem2, sem3):
    wid = lax.axis_index("s") * NC + lax.axis_index("c")

    pltpu.sync_copy(idx_hbm.at[pl.ds(wid * NCHUNK, NCHUNK)], idx_v)

    bufs = ((rows0_v, sem0), (rows1_v, sem1), (rows2_v, sem2),
            (rows3_v, sem3))
    NBUF = len(bufs)

    def gather(c, buf, sem):
        pltpu.async_copy(table_hbm.at[idx_v.at[jnp.minimum(c, NCHUNK - 1)]],
                         buf, sem)

    def drain(buf, sem):
        pltpu.make_async_copy(table_hbm.at[idx_v.at[0]], buf, sem).wait()

    def accumulate(c, buf):
        for i in range(CB):
            def bag_body(r, accs):
                return tuple(
                    accs[j] + buf[i * HIST + r, pl.ds(j * LANES, LANES)]
                    for j in range(EV)
                )
            accs = lax.fori_loop(
                0, HIST, bag_body,
                tuple(jnp.zeros((LANES,), jnp.float32) for _ in range(EV)),
                unroll=10,
            )
            for j in range(EV):
                pool_v[c * CB + i, pl.ds(j * LANES, LANES)] = accs[j]

    gather(0, *bufs[0])
    gather(1, *bufs[1])

    def ring_body(p, carry):
        a = p * NBUF
        for q in range(NBUF):
            gather(a + q + 2, *bufs[(q + 2) % NBUF])
            drain(*bufs[q])
            accumulate(a + q, bufs[q][0])
        return carry

    lax.fori_loop(0, NCHUNK // NBUF, ring_body, 0)
    drain(*bufs[0])
    drain(*bufs[1])

    pltpu.sync_copy(pool_v, out_hbm.at[pl.ds(wid * ROWS_PER_W, ROWS_PER_W)])


def _mlp_body(x_ref, w1_ref, b1_ref, w2_ref, b2_ref, o_ref):
    x = x_ref[...] * (1.0 / HIST)
    h = lax.dot_general(
        x, w1_ref[...], (((1,), (1,)), ((), ())),
        preferred_element_type=jnp.float32,
    ) + b1_ref[...]
    o_ref[...] = lax.dot_general(
        h, w2_ref[...], (((1,), (1,)), ((), ())),
        preferred_element_type=jnp.float32,
    ) + b2_ref[...]


_mlp = pl.pallas_call(
    _mlp_body,
    out_shape=jax.ShapeDtypeStruct((BATCH, OUT), jnp.float32),
)


@jax.jit
def kernel(input_batch, emb_table, W1, b1, W2, b2):
    idx2d = input_batch.astype(jnp.int32).reshape(BATCH // CB, CHUNK_IDX)
    pooled_sum = _embbag_sum(idx2d, emb_table)
    return _mlp(pooled_sum, W1, b1.reshape(1, HIDDEN), W2, b2.reshape(1, OUT))


# speedup vs baseline: 1.3053x; 1.0176x over previous
import functools

import jax
import jax.numpy as jnp
from jax import lax
from jax.experimental import pallas as pl
from jax.experimental.pallas import tpu as pltpu
from jax.experimental.pallas import tpu_sc as plsc

VOCAB = 100000
EMBED = 128
HIDDEN = 512
OUT = 256
BATCH = 4096
HIST = 50

NC = 2
NS = 16
NW = NC * NS
ROWS_PER_W = BATCH // NW
CB = 2
CHUNK_IDX = CB * HIST
NCHUNK = ROWS_PER_W // CB
LANES = 16
EV = EMBED // LANES

_sc_mesh = plsc.VectorSubcoreMesh(
    core_axis_name="c", subcore_axis_name="s", num_cores=NC, num_subcores=NS
)


@functools.partial(
    pl.kernel,
    out_type=jax.ShapeDtypeStruct((BATCH, EMBED), jnp.float32),
    mesh=_sc_mesh,
    scratch_types=[
        pltpu.VMEM((NCHUNK, CHUNK_IDX), jnp.int32),
        pltpu.VMEM((CHUNK_IDX, EMBED), jnp.float32),
        pltpu.VMEM((CHUNK_IDX, EMBED), jnp.float32),
        pltpu.VMEM((CHUNK_IDX, EMBED), jnp.float32),
        pltpu.VMEM((CHUNK_IDX, EMBED), jnp.float32),
        pltpu.VMEM((ROWS_PER_W, EMBED), jnp.float32),
        pltpu.SemaphoreType.DMA,
        pltpu.SemaphoreType.DMA,
        pltpu.SemaphoreType.DMA,
        pltpu.SemaphoreType.DMA,
    ],
)
def _embbag_sum(idx_hbm, table_hbm, out_hbm, idx_v, rows0_v, rows1_v,
                rows2_v, rows3_v, pool_v, sem0, sem1, sem2, sem3):
    wid = lax.axis_index("s") * NC + lax.axis_index("c")

    pltpu.sync_copy(idx_hbm.at[pl.ds(wid * NCHUNK, NCHUNK)], idx_v)

    bufs = ((rows0_v, sem0), (rows1_v, sem1), (rows2_v, sem2),
            (rows3_v, sem3))
    NBUF = len(bufs)

    def gather(c, buf, sem):
        pltpu.async_copy(table_hbm.at[idx_v.at[jnp.minimum(c, NCHUNK - 1)]],
                         buf, sem)

    def drain(buf, sem):
        pltpu.make_async_copy(table_hbm.at[idx_v.at[0]], buf, sem).wait()

    def accumulate(c, buf):
        for i in range(CB):
            def bag_body(r, accs):
                return tuple(
                    accs[j] + buf[i * HIST + r, pl.ds(j * LANES, LANES)]
                    for j in range(EV)
                )
            accs = lax.fori_loop(
                0, HIST, bag_body,
                tuple(jnp.zeros((LANES,), jnp.float32) for _ in range(EV)),
                unroll=5,
            )
            for j in range(EV):
                pool_v[c * CB + i, pl.ds(j * LANES, LANES)] = accs[j]

    gather(0, *bufs[0])
    gather(1, *bufs[1])

    def ring_body(p, carry):
        a = p * NBUF
        for q in range(NBUF):
            gather(a + q + 2, *bufs[(q + 2) % NBUF])
            drain(*bufs[q])
            accumulate(a + q, bufs[q][0])
        return carry

    lax.fori_loop(0, NCHUNK // NBUF, ring_body, 0)
    drain(*bufs[0])
    drain(*bufs[1])

    pltpu.sync_copy(pool_v, out_hbm.at[pl.ds(wid * ROWS_PER_W, ROWS_PER_W)])


def _mlp_body(x_ref, w1_ref, b1_ref, w2_ref, b2_ref, o_ref):
    x = x_ref[...] * (1.0 / HIST)
    h = lax.dot_general(
        x, w1_ref[...], (((1,), (1,)), ((), ())),
        preferred_element_type=jnp.float32,
    ) + b1_ref[...]
    o_ref[...] = lax.dot_general(
        h, w2_ref[...], (((1,), (1,)), ((), ())),
        preferred_element_type=jnp.float32,
    ) + b2_ref[...]


_mlp = pl.pallas_call(
    _mlp_body,
    out_shape=jax.ShapeDtypeStruct((BATCH, OUT), jnp.float32),
)


@jax.jit
def kernel(input_batch, emb_table, W1, b1, W2, b2):
    idx2d = input_batch.astype(jnp.int32).reshape(BATCH // CB, CHUNK_IDX)
    pooled_sum = _embbag_sum(idx2d, emb_table)
    return _mlp(pooled_sum, W1, b1.reshape(1, HIDDEN), W2, b2.reshape(1, OUT))


# speedup vs baseline: 1.3738x; 1.0525x over previous
import functools

import jax
import jax.numpy as jnp
from jax import lax
from jax.experimental import pallas as pl
from jax.experimental.pallas import tpu as pltpu
from jax.experimental.pallas import tpu_sc as plsc

VOCAB = 100000
EMBED = 128
HIDDEN = 512
OUT = 256
BATCH = 4096
HIST = 50

NC = 2
NS = 16
NW = NC * NS
ROWS_PER_W = BATCH // NW
CB = 2
CHUNK_IDX = CB * HIST
NCHUNK = ROWS_PER_W // CB
LANES = 16
EV = EMBED // LANES
NBUF = 8
DEPTH = 4

_sc_mesh = plsc.VectorSubcoreMesh(
    core_axis_name="c", subcore_axis_name="s", num_cores=NC, num_subcores=NS
)


@functools.partial(
    pl.kernel,
    out_type=jax.ShapeDtypeStruct((BATCH, EMBED), jnp.float32),
    mesh=_sc_mesh,
    scratch_types=[
        pltpu.VMEM((NCHUNK, CHUNK_IDX), jnp.int32),
        pltpu.VMEM((ROWS_PER_W, EMBED), jnp.float32),
    ]
    + [pltpu.VMEM((CHUNK_IDX, EMBED), jnp.float32) for _ in range(NBUF)]
    + [pltpu.SemaphoreType.DMA for _ in range(NBUF)],
)
def _embbag_sum(idx_hbm, table_hbm, out_hbm, idx_v, pool_v, *bufs_and_sems):
    bufs = tuple(zip(bufs_and_sems[:NBUF], bufs_and_sems[NBUF:]))
    wid = lax.axis_index("s") * NC + lax.axis_index("c")

    pltpu.sync_copy(idx_hbm.at[pl.ds(wid * NCHUNK, NCHUNK)], idx_v)

    def gather(c, buf, sem):
        pltpu.async_copy(table_hbm.at[idx_v.at[jnp.minimum(c, NCHUNK - 1)]],
                         buf, sem)

    def drain(buf, sem):
        pltpu.make_async_copy(table_hbm.at[idx_v.at[0]], buf, sem).wait()

    def accumulate(c, buf):
        for i in range(CB):
            def bag_body(r, accs):
                return tuple(
                    accs[j] + buf[i * HIST + r, pl.ds(j * LANES, LANES)]
                    for j in range(EV)
                )
            accs = lax.fori_loop(
                0, HIST, bag_body,
                tuple(jnp.zeros((LANES,), jnp.float32) for _ in range(EV)),
                unroll=5,
            )
            for j in range(EV):
                pool_v[c * CB + i, pl.ds(j * LANES, LANES)] = accs[j]

    for d in range(DEPTH):
        gather(d, *bufs[d])

    def ring_body(p, carry):
        a = p * NBUF
        for q in range(NBUF):
            drain(*bufs[q])
            gather(a + q + DEPTH, *bufs[(q + DEPTH) % NBUF])
            accumulate(a + q, bufs[q][0])
        return carry

    lax.fori_loop(0, NCHUNK // NBUF, ring_body, 0)
    for d in range(DEPTH):
        drain(*bufs[d])

    pltpu.sync_copy(pool_v, out_hbm.at[pl.ds(wid * ROWS_PER_W, ROWS_PER_W)])


def _mlp_body(x_ref, w1_ref, b1_ref, w2_ref, b2_ref, o_ref):
    x = x_ref[...] * (1.0 / HIST)
    h = lax.dot_general(
        x, w1_ref[...], (((1,), (1,)), ((), ())),
        preferred_element_type=jnp.float32,
    ) + b1_ref[...]
    o_ref[...] = lax.dot_general(
        h, w2_ref[...], (((1,), (1,)), ((), ())),
        preferred_element_type=jnp.float32,
    ) + b2_ref[...]


_mlp = pl.pallas_call(
    _mlp_body,
    out_shape=jax.ShapeDtypeStruct((BATCH, OUT), jnp.float32),
)


@jax.jit
def kernel(input_batch, emb_table, W1, b1, W2, b2):
    idx2d = input_batch.astype(jnp.int32).reshape(BATCH // CB, CHUNK_IDX)
    pooled_sum = _embbag_sum(idx2d, emb_table)
    return _mlp(pooled_sum, W1, b1.reshape(1, HIDDEN), W2, b2.reshape(1, OUT))
